# Initial kernel scaffold; baseline (speedup 1.0000x reference)
import functools

import jax
import jax.numpy as jnp
from jax.experimental import pallas as pl

_N = 4
_D = 64
_H = 4
_HD = _D // _H
_SCALE = _HD ** -0.5

_BLK_S = 256
_BLK_R = _BLK_S * _N


def _ln(x, g, b, eps=1e-6):
    m = jnp.mean(x, axis=-1, keepdims=True)
    v = jnp.mean(jnp.square(x - m), axis=-1, keepdims=True)
    return (x - m) * jax.lax.rsqrt(v + eps) * g + b


def _dot(a, b):
    return jnp.dot(a, b, preferred_element_type=jnp.float32)


def _group_shift(x, delta, rowmod):
    if delta == 0:
        return x
    a = jnp.roll(x, -delta, axis=0)
    b = jnp.roll(x, _N - delta, axis=0)
    return jnp.where(rowmod < (_N - delta), a, b)


def _body(x1_ref, x2_ref,
          fcd_w_ref, fcd_b_ref, fct_w_ref, fct_b_ref,
          d1_wq, d1_wk, d1_wv, d1_pw, d1_pb, d1_f1w, d1_f1b, d1_f2w, d1_f2b,
          d1_n1g, d1_n1b, d1_n2g, d1_n2b,
          d2_wq, d2_wk, d2_wv, d2_pw, d2_pb, d2_f1w, d2_f1b, d2_f2w, d2_f2b,
          d2_n1g, d2_n1b, d2_n2g, d2_n2b,
          n1g_ref, n1b_ref, n2g_ref, n2b_ref, w1_ref, w2_ref, fc3b_ref,
          g_ref, out_ref):
    rowmod = jax.lax.broadcasted_iota(jnp.int32, (_BLK_R, 1), 0) % _N
    mh = (jax.lax.broadcasted_iota(jnp.int32, (_D, _H), 0) // _HD
          == jax.lax.broadcasted_iota(jnp.int32, (_D, _H), 1)
          ).astype(jnp.float32)
    ex = (jax.lax.broadcasted_iota(jnp.int32, (_H, _D), 0)
          == jax.lax.broadcasted_iota(jnp.int32, (_H, _D), 1) // _HD
          ).astype(jnp.float32)

    f1 = jax.nn.relu(_dot(x1_ref[...], fcd_w_ref[...]) + fcd_b_ref[...])
    f2 = jax.nn.relu(_dot(x2_ref[...], fct_w_ref[...]) + fct_b_ref[...])

    def branch(x, y, wq, wk, wv, pw, pb, f1w, f1b, f2w, f2b,
               ng1, nb1, ng2, nb2):
        k = _dot(x, wk)
        v = _dot(x, wv)
        kd = [_group_shift(k, d, rowmod) for d in range(_N)]
        vd = [_group_shift(v, d, rowmod) for d in range(_N)]
        for _ in range(4):
            q = _dot(_ln(y, ng1, nb1), wq)
            s = [_dot(q * kd[d], mh) * _SCALE for d in range(_N)]
            m = functools.reduce(jnp.maximum, s)
            e = [jnp.exp(si - m) for si in s]
            inv = 1.0 / (e[0] + e[1] + e[2] + e[3])
            att = sum(_dot(e[d] * inv, ex) * vd[d] for d in range(_N))
            ca = _dot(att, pw) + pb
            h = jax.nn.gelu(_dot(_ln(y, ng2, nb2), f1w) + f1b,
                            approximate=False)
            y = y + ca + (_dot(h, f2w) + f2b)
        return y

    o1 = branch(f1, f2,
                d1_wq[...], d1_wk[...], d1_wv[...], d1_pw[...], d1_pb[...],
                d1_f1w[...], d1_f1b[...], d1_f2w[...], d1_f2b[...],
                d1_n1g[...], d1_n1b[...], d1_n2g[...], d1_n2b[...])
    o2 = branch(f2, f1,
                d2_wq[...], d2_wk[...], d2_wv[...], d2_pw[...], d2_pb[...],
                d2_f1w[...], d2_f1b[...], d2_f2w[...], d2_f2b[...],
                d2_n1g[...], d2_n1b[...], d2_n2g[...], d2_n2b[...])
    o1 = _ln(o1, n1g_ref[...], n1b_ref[...])
    o2 = _ln(o2, n2g_ref[...], n2b_ref[...])

    g_ref[...] = jnp.concatenate([o1, o2], axis=1)

    z = _dot(o1, w1_ref[...]) + _dot(o2, w2_ref[...])
    lane4 = jax.lax.broadcasted_iota(jnp.int32, (_BLK_R, _N), 1)
    zm = jnp.where(lane4 == rowmod, z, 0.0)
    zs = zm.reshape(_BLK_S, _N, _N).sum(axis=1).sum(axis=1, keepdims=True)
    out_ref[...] = jax.nn.sigmoid(zs + fc3b_ref[0, 0])


def kernel(feature1, feature2, fc_d_w, fc_d_b, fc_t_w, fc_t_b, dec1, dec2,
           norm1_g, norm1_b, norm2_g, norm2_b, fc3_w, fc3_b):
    B, N, F = feature1.shape
    R = B * N
    x1 = feature1.reshape(R, F)
    x2 = feature2.reshape(R, F)

    def prep_dec(p):
        qkv = p['qkv_w']
        return [qkv[:, :_D], qkv[:, _D:2 * _D], qkv[:, 2 * _D:],
                p['proj_w'], p['proj_b'].reshape(1, -1),
                p['fc1_w'], p['fc1_b'].reshape(1, -1),
                p['fc2_w'], p['fc2_b'].reshape(1, -1),
                p['n1_g'].reshape(1, -1), p['n1_b'].reshape(1, -1),
                p['n2_g'].reshape(1, -1), p['n2_b'].reshape(1, -1)]

    fc3r = fc3_w.reshape(N, 2, _D)
    w1 = fc3r[:, 0, :].T
    w2 = fc3r[:, 1, :].T
    weights = ([fc_d_w, fc_d_b.reshape(1, -1), fc_t_w, fc_t_b.reshape(1, -1)]
               + prep_dec(dec1) + prep_dec(dec2)
               + [norm1_g.reshape(1, -1), norm1_b.reshape(1, -1),
                  norm2_g.reshape(1, -1), norm2_b.reshape(1, -1),
                  w1, w2, fc3_b.reshape(1, 1)])

    grid = R // _BLK_R
    in_specs = [pl.BlockSpec((_BLK_R, F), lambda i: (i, 0)),
                pl.BlockSpec((_BLK_R, F), lambda i: (i, 0))]
    in_specs += [pl.BlockSpec(w.shape, lambda i: (0, 0)) for w in weights]
    out_shape = [jax.ShapeDtypeStruct((R, 2 * _D), jnp.float32),
                 jax.ShapeDtypeStruct((B, 1), jnp.float32)]
    out_specs = [pl.BlockSpec((_BLK_R, 2 * _D), lambda i: (i, 0)),
                 pl.BlockSpec((_BLK_S, 1), lambda i: (i, 0))]

    g, out = pl.pallas_call(
        _body, grid=(grid,), in_specs=in_specs, out_specs=out_specs,
        out_shape=out_shape)(x1, x2, *weights)
    return (out.reshape(B), g.reshape(B, N * 2 * _D))


# speedup vs baseline: 4.2368x; 4.2368x over previous
import functools

import jax
import jax.numpy as jnp
from jax.experimental import pallas as pl

_N = 4
_D = 64
_H = 4
_HD = _D // _H
_SCALE = _HD ** -0.5

_BLK_S = 256
_BLK_R = _BLK_S * _N


def _ln(x, g, b, eps=1e-6):
    m = jnp.mean(x, axis=-1, keepdims=True)
    v = jnp.mean(jnp.square(x - m), axis=-1, keepdims=True)
    return (x - m) * jax.lax.rsqrt(v + eps) * g + b


def _dot(a, b):
    return jnp.dot(a, b, preferred_element_type=jnp.float32)


def _group_shift(x, delta, rowmod):
    if delta == 0:
        return x
    a = jnp.roll(x, -delta, axis=0)
    b = jnp.roll(x, _N - delta, axis=0)
    return jnp.where(rowmod < (_N - delta), a, b)


def _body(x1_ref, x2_ref,
          fcd_w_ref, fcd_b_ref, fct_w_ref, fct_b_ref,
          d1_wq, d1_wk, d1_wv, d1_pw, d1_pb, d1_f1w, d1_f1b, d1_f2w, d1_f2b,
          d1_n1g, d1_n1b, d1_n2g, d1_n2b,
          d2_wq, d2_wk, d2_wv, d2_pw, d2_pb, d2_f1w, d2_f1b, d2_f2w, d2_f2b,
          d2_n1g, d2_n1b, d2_n2g, d2_n2b,
          n1g_ref, n1b_ref, n2g_ref, n2b_ref, w1_ref, w2_ref, fc3b_ref,
          g_ref, out_ref):
    rowmod = jax.lax.broadcasted_iota(jnp.int32, (_BLK_R, 1), 0) % _N
    mh = (jax.lax.broadcasted_iota(jnp.int32, (_D, _H), 0) // _HD
          == jax.lax.broadcasted_iota(jnp.int32, (_D, _H), 1)
          ).astype(jnp.float32)
    ex = (jax.lax.broadcasted_iota(jnp.int32, (_H, _D), 0)
          == jax.lax.broadcasted_iota(jnp.int32, (_H, _D), 1) // _HD
          ).astype(jnp.float32)

    f1 = jax.nn.relu(_dot(x1_ref[...], fcd_w_ref[...]) + fcd_b_ref[...])
    f2 = jax.nn.relu(_dot(x2_ref[...], fct_w_ref[...]) + fct_b_ref[...])

    def branch(x, y, wq, wk, wv, pw, pb, f1w, f1b, f2w, f2b,
               ng1, nb1, ng2, nb2):
        k = _dot(x, wk)
        v = _dot(x, wv)
        kd = [_group_shift(k, d, rowmod) for d in range(_N)]
        vd = [_group_shift(v, d, rowmod) for d in range(_N)]
        for _ in range(4):
            q = _dot(_ln(y, ng1, nb1), wq)
            s = [_dot(q * kd[d], mh) * _SCALE for d in range(_N)]
            m = functools.reduce(jnp.maximum, s)
            e = [jnp.exp(si - m) for si in s]
            inv = 1.0 / (e[0] + e[1] + e[2] + e[3])
            att = sum(_dot(e[d] * inv, ex) * vd[d] for d in range(_N))
            ca = _dot(att, pw) + pb
            u = _dot(_ln(y, ng2, nb2), f1w) + f1b
            h = 0.5 * u * (1.0 + jax.lax.erf(u * 0.7071067811865476))
            y = y + ca + (_dot(h, f2w) + f2b)
        return y

    o1 = branch(f1, f2,
                d1_wq[...], d1_wk[...], d1_wv[...], d1_pw[...], d1_pb[...],
                d1_f1w[...], d1_f1b[...], d1_f2w[...], d1_f2b[...],
                d1_n1g[...], d1_n1b[...], d1_n2g[...], d1_n2b[...])
    o2 = branch(f2, f1,
                d2_wq[...], d2_wk[...], d2_wv[...], d2_pw[...], d2_pb[...],
                d2_f1w[...], d2_f1b[...], d2_f2w[...], d2_f2b[...],
                d2_n1g[...], d2_n1b[...], d2_n2g[...], d2_n2b[...])
    o1 = _ln(o1, n1g_ref[...], n1b_ref[...])
    o2 = _ln(o2, n2g_ref[...], n2b_ref[...])

    g_ref[...] = jnp.concatenate([o1, o2], axis=1)

    z = _dot(o1, w1_ref[...]) + _dot(o2, w2_ref[...])
    lane4 = jax.lax.broadcasted_iota(jnp.int32, (_BLK_R, _N), 1)
    zm = jnp.where(lane4 == rowmod, z, 0.0)
    zs = zm.reshape(_BLK_S, _N, _N).sum(axis=1).sum(axis=1, keepdims=True)
    out_ref[...] = jax.nn.sigmoid(zs + fc3b_ref[0, 0])


def kernel(feature1, feature2, fc_d_w, fc_d_b, fc_t_w, fc_t_b, dec1, dec2,
           norm1_g, norm1_b, norm2_g, norm2_b, fc3_w, fc3_b):
    B, N, F = feature1.shape
    R = B * N
    x1 = feature1.reshape(R, F)
    x2 = feature2.reshape(R, F)

    def prep_dec(p):
        qkv = p['qkv_w']
        return [qkv[:, :_D], qkv[:, _D:2 * _D], qkv[:, 2 * _D:],
                p['proj_w'], p['proj_b'].reshape(1, -1),
                p['fc1_w'], p['fc1_b'].reshape(1, -1),
                p['fc2_w'], p['fc2_b'].reshape(1, -1),
                p['n1_g'].reshape(1, -1), p['n1_b'].reshape(1, -1),
                p['n2_g'].reshape(1, -1), p['n2_b'].reshape(1, -1)]

    fc3r = fc3_w.reshape(N, 2, _D)
    w1 = fc3r[:, 0, :].T
    w2 = fc3r[:, 1, :].T
    weights = ([fc_d_w, fc_d_b.reshape(1, -1), fc_t_w, fc_t_b.reshape(1, -1)]
               + prep_dec(dec1) + prep_dec(dec2)
               + [norm1_g.reshape(1, -1), norm1_b.reshape(1, -1),
                  norm2_g.reshape(1, -1), norm2_b.reshape(1, -1),
                  w1, w2, fc3_b.reshape(1, 1)])

    grid = R // _BLK_R
    in_specs = [pl.BlockSpec((_BLK_R, F), lambda i: (i, 0)),
                pl.BlockSpec((_BLK_R, F), lambda i: (i, 0))]
    in_specs += [pl.BlockSpec(w.shape, lambda i: (0, 0)) for w in weights]
    out_shape = [jax.ShapeDtypeStruct((R, 2 * _D), jnp.float32),
                 jax.ShapeDtypeStruct((B, 1), jnp.float32)]
    out_specs = [pl.BlockSpec((_BLK_R, 2 * _D), lambda i: (i, 0)),
                 pl.BlockSpec((_BLK_S, 1), lambda i: (i, 0))]

    g, out = pl.pallas_call(
        _body, grid=(grid,), in_specs=in_specs, out_specs=out_specs,
        out_shape=out_shape)(x1, x2, *weights)
    return (out.reshape(B), g.reshape(B, N * 2 * _D))


# speedup vs baseline: 5.2771x; 1.2456x over previous
import functools

import jax
import jax.numpy as jnp
from jax.experimental import pallas as pl

_N = 4
_D = 64
_H = 4
_HD = _D // _H
_SCALE = _HD ** -0.5
_P = 2 * _D

_BLK_S = 256
_BLK_R = _BLK_S * _N


def _dot(a, b):
    return jnp.dot(a, b, preferred_element_type=jnp.float32)


def _group_shift(x, delta, rowmod):
    if delta == 0:
        return x
    a = jnp.roll(x, -delta, axis=0)
    b = jnp.roll(x, _N - delta, axis=0)
    return jnp.where(rowmod < (_N - delta), a, b)


def _body(x1_ref, x2_ref, wd_ref, bd_ref, wt_ref, bt_ref,
          wk_ref, wv_ref, wq_ref, bq_ref, pw_ref, pb_ref,
          f1w_ref, f1b_ref, f2w_ref, f2b_ref,
          gf_ref, bf_ref, w12_ref, fc3b_ref,
          g_ref, out_ref):
    rowmod = jax.lax.broadcasted_iota(jnp.int32, (_BLK_R, 1), 0) % _N
    ci = jax.lax.broadcasted_iota(jnp.int32, (_P, _P), 0)
    cj = jax.lax.broadcasted_iota(jnp.int32, (_P, _P), 1)
    gmat = jnp.where(ci // _HD == cj // _HD, _SCALE, 0.0).astype(jnp.float32)
    mmat = jnp.where(ci // _D == cj // _D, 1.0 / _D, 0.0).astype(jnp.float32)

    def ln_norm(y):
        m = _dot(y, mmat)
        s2 = _dot(y * y, mmat)
        return (y - m) * jax.lax.rsqrt(s2 - m * m + 1e-6)

    f1 = jax.nn.relu(_dot(x1_ref[...], wd_ref[...]) + bd_ref[...])
    f2 = jax.nn.relu(_dot(x2_ref[...], wt_ref[...]) + bt_ref[...])
    fp = jnp.concatenate([f1, f2], axis=1)
    y = jnp.concatenate([f2, f1], axis=1)

    k = _dot(fp, wk_ref[...])
    v = _dot(fp, wv_ref[...])
    kd = [_group_shift(k, d, rowmod) for d in range(_N)]
    vd = [_group_shift(v, d, rowmod) for d in range(_N)]

    wq, bq = wq_ref[...], bq_ref[...]
    pw, pb = pw_ref[...], pb_ref[...]
    f1w, f1b = f1w_ref[...], f1b_ref[...]
    f2w, f2b = f2w_ref[...], f2b_ref[...]
    for _ in range(4):
        xn = ln_norm(y)
        q = _dot(xn, wq) + bq
        s = [_dot(q * kd[d], gmat) for d in range(_N)]
        m = functools.reduce(jnp.maximum, s)
        e = [jnp.exp(si - m) for si in s]
        inv = 1.0 / (e[0] + e[1] + e[2] + e[3])
        att = (e[0] * vd[0] + e[1] * vd[1] + e[2] * vd[2] + e[3] * vd[3]) * inv
        ca = _dot(att, pw) + pb
        u = _dot(xn, f1w) + f1b
        h = 0.5 * u * (1.0 + jax.lax.erf(u * 0.7071067811865476))
        y = y + ca + (_dot(h, f2w) + f2b)

    o = ln_norm(y) * gf_ref[...] + bf_ref[...]
    g_ref[...] = o

    z = _dot(o, w12_ref[...])
    lane4 = jax.lax.broadcasted_iota(jnp.int32, (_BLK_R, _N), 1)
    zm = jnp.where(lane4 == rowmod, z, 0.0)
    zs = zm.reshape(_BLK_S, _N, _N).sum(axis=1).sum(axis=1, keepdims=True)
    out_ref[...] = jax.nn.sigmoid(zs + fc3b_ref[0, 0])


def _blockdiag(a, b):
    ra, ca = a.shape
    rb, cb = b.shape
    return jnp.concatenate([
        jnp.concatenate([a, jnp.zeros((ra, cb), a.dtype)], axis=1),
        jnp.concatenate([jnp.zeros((rb, ca), b.dtype), b], axis=1)], axis=0)


def kernel(feature1, feature2, fc_d_w, fc_d_b, fc_t_w, fc_t_b, dec1, dec2,
           norm1_g, norm1_b, norm2_g, norm2_b, fc3_w, fc3_b):
    B, N, F = feature1.shape
    R = B * N
    x1 = feature1.reshape(R, F)
    x2 = feature2.reshape(R, F)

    def slices(p):
        qkv = p['qkv_w']
        return qkv[:, :_D], qkv[:, _D:2 * _D], qkv[:, 2 * _D:]

    wq1, wk1, wv1 = slices(dec1)
    wq2, wk2, wv2 = slices(dec2)
    wq1f = dec1['n1_g'][:, None] * wq1
    wq2f = dec2['n1_g'][:, None] * wq2
    bq = jnp.concatenate([dec1['n1_b'] @ wq1, dec2['n1_b'] @ wq2])
    f1w1 = dec1['n2_g'][:, None] * dec1['fc1_w']
    f1w2 = dec2['n2_g'][:, None] * dec2['fc1_w']
    f1b = jnp.concatenate([dec1['n2_b'] @ dec1['fc1_w'] + dec1['fc1_b'],
                           dec2['n2_b'] @ dec2['fc1_w'] + dec2['fc1_b']])

    fc3r = fc3_w.reshape(N, 2, _D)
    w12 = jnp.concatenate([fc3r[:, 0, :].T, fc3r[:, 1, :].T], axis=0)

    weights = [
        fc_d_w, fc_d_b.reshape(1, -1), fc_t_w, fc_t_b.reshape(1, -1),
        _blockdiag(wk1, wk2), _blockdiag(wv1, wv2),
        _blockdiag(wq1f, wq2f), bq.reshape(1, -1),
        _blockdiag(dec1['proj_w'], dec2['proj_w']),
        jnp.concatenate([dec1['proj_b'], dec2['proj_b']]).reshape(1, -1),
        _blockdiag(f1w1, f1w2), f1b.reshape(1, -1),
        _blockdiag(dec1['fc2_w'], dec2['fc2_w']),
        jnp.concatenate([dec1['fc2_b'], dec2['fc2_b']]).reshape(1, -1),
        jnp.concatenate([norm1_g, norm2_g]).reshape(1, -1),
        jnp.concatenate([norm1_b, norm2_b]).reshape(1, -1),
        w12, fc3_b.reshape(1, 1),
    ]

    grid = R // _BLK_R
    in_specs = [pl.BlockSpec((_BLK_R, F), lambda i: (i, 0)),
                pl.BlockSpec((_BLK_R, F), lambda i: (i, 0))]
    in_specs += [pl.BlockSpec(w.shape, lambda i: (0, 0)) for w in weights]
    out_shape = [jax.ShapeDtypeStruct((R, _P), jnp.float32),
                 jax.ShapeDtypeStruct((B, 1), jnp.float32)]
    out_specs = [pl.BlockSpec((_BLK_R, _P), lambda i: (i, 0)),
                 pl.BlockSpec((_BLK_S, 1), lambda i: (i, 0))]

    g, out = pl.pallas_call(
        _body, grid=(grid,), in_specs=in_specs, out_specs=out_specs,
        out_shape=out_shape)(x1, x2, *weights)
    return (out.reshape(B), g.reshape(B, N * _P))


# speedup vs baseline: 5.9699x; 1.1313x over previous
import functools

import jax
import jax.numpy as jnp
from jax.experimental import pallas as pl

_N = 4
_D = 64
_H = 4
_HD = _D // _H
_SCALE = _HD ** -0.5
_P = 2 * _D

_BLK_S = 256
_BLK_R = _BLK_S * _N


def _dot(a, b):
    return jnp.dot(a, b, preferred_element_type=jnp.float32)


def _group_shift(x, delta, rowmod):
    if delta == 0:
        return x
    a = jnp.roll(x, -delta, axis=0)
    b = jnp.roll(x, _N - delta, axis=0)
    return jnp.where(rowmod < (_N - delta), a, b)


def _body(x1_ref, x2_ref, wd_ref, bd_ref, wt_ref, bt_ref,
          wk_ref, wv_ref, wq_ref, bq_ref, pw_ref, pb_ref,
          f1w_ref, f1b_ref, f2w_ref, f2b_ref,
          gf_ref, bf_ref, w12_ref, fc3b_ref,
          g_ref, out_ref):
    rowmod = jax.lax.broadcasted_iota(jnp.int32, (_BLK_R, 1), 0) % _N
    ci = jax.lax.broadcasted_iota(jnp.int32, (_P, _P), 0)
    cj = jax.lax.broadcasted_iota(jnp.int32, (_P, _P), 1)
    gmat = jnp.where(ci // _HD == cj // _HD, _SCALE, 0.0).astype(jnp.float32)
    mmat = jnp.where(ci // _D == cj // _D, 1.0 / _D, 0.0).astype(jnp.float32)

    def ln_norm(y):
        m = _dot(y, mmat)
        s2 = _dot(y * y, mmat)
        return (y - m) * jax.lax.rsqrt(s2 - m * m + 1e-6)

    x1 = x1_ref[...].reshape(_BLK_R, x1_ref.shape[-1])
    x2 = x2_ref[...].reshape(_BLK_R, x2_ref.shape[-1])
    f1 = jax.nn.relu(_dot(x1, wd_ref[...]) + bd_ref[...])
    f2 = jax.nn.relu(_dot(x2, wt_ref[...]) + bt_ref[...])
    fp = jnp.concatenate([f1, f2], axis=1)
    y = jnp.concatenate([f2, f1], axis=1)

    k = _dot(fp, wk_ref[...])
    v = _dot(fp, wv_ref[...])
    kd = [_group_shift(k, d, rowmod) for d in range(_N)]
    vd = [_group_shift(v, d, rowmod) for d in range(_N)]

    wq, bq = wq_ref[...], bq_ref[...]
    pw, pb = pw_ref[...], pb_ref[...]
    f1w, f1b = f1w_ref[...], f1b_ref[...]
    f2w, f2b = f2w_ref[...], f2b_ref[...]
    for _ in range(4):
        xn = ln_norm(y)
        q = _dot(xn, wq) + bq
        s = [_dot(q * kd[d], gmat) for d in range(_N)]
        m = functools.reduce(jnp.maximum, s)
        e = [jnp.exp(si - m) for si in s]
        inv = 1.0 / (e[0] + e[1] + e[2] + e[3])
        att = (e[0] * vd[0] + e[1] * vd[1] + e[2] * vd[2] + e[3] * vd[3]) * inv
        ca = _dot(att, pw) + pb
        u = _dot(xn, f1w) + f1b
        h = 0.5 * u * (1.0 + jax.lax.erf(u * 0.7071067811865476))
        y = y + ca + (_dot(h, f2w) + f2b)

    o = ln_norm(y) * gf_ref[...] + bf_ref[...]
    g_ref[...] = o

    z = _dot(o, w12_ref[...])
    lane4 = jax.lax.broadcasted_iota(jnp.int32, (_BLK_R, _N), 1)
    zm = jnp.where(lane4 == rowmod, z, 0.0)
    zs = zm.reshape(_BLK_S, _N, _N).sum(axis=1).sum(axis=1, keepdims=True)
    out_ref[...] = jax.nn.sigmoid(zs + fc3b_ref[0, 0])


def _blockdiag(a, b):
    ra, ca = a.shape
    rb, cb = b.shape
    return jnp.concatenate([
        jnp.concatenate([a, jnp.zeros((ra, cb), a.dtype)], axis=1),
        jnp.concatenate([jnp.zeros((rb, ca), b.dtype), b], axis=1)], axis=0)


def kernel(feature1, feature2, fc_d_w, fc_d_b, fc_t_w, fc_t_b, dec1, dec2,
           norm1_g, norm1_b, norm2_g, norm2_b, fc3_w, fc3_b):
    B, N, F = feature1.shape
    R = B * N

    def slices(p):
        qkv = p['qkv_w']
        return qkv[:, :_D], qkv[:, _D:2 * _D], qkv[:, 2 * _D:]

    wq1, wk1, wv1 = slices(dec1)
    wq2, wk2, wv2 = slices(dec2)
    wq1f = dec1['n1_g'][:, None] * wq1
    wq2f = dec2['n1_g'][:, None] * wq2
    bq = jnp.concatenate([dec1['n1_b'] @ wq1, dec2['n1_b'] @ wq2])
    f1w1 = dec1['n2_g'][:, None] * dec1['fc1_w']
    f1w2 = dec2['n2_g'][:, None] * dec2['fc1_w']
    f1b = jnp.concatenate([dec1['n2_b'] @ dec1['fc1_w'] + dec1['fc1_b'],
                           dec2['n2_b'] @ dec2['fc1_w'] + dec2['fc1_b']])

    fc3r = fc3_w.reshape(N, 2, _D)
    w12 = jnp.concatenate([fc3r[:, 0, :].T, fc3r[:, 1, :].T], axis=0)

    weights = [
        fc_d_w, fc_d_b.reshape(1, -1), fc_t_w, fc_t_b.reshape(1, -1),
        _blockdiag(wk1, wk2), _blockdiag(wv1, wv2),
        _blockdiag(wq1f, wq2f), bq.reshape(1, -1),
        _blockdiag(dec1['proj_w'], dec2['proj_w']),
        jnp.concatenate([dec1['proj_b'], dec2['proj_b']]).reshape(1, -1),
        _blockdiag(f1w1, f1w2), f1b.reshape(1, -1),
        _blockdiag(dec1['fc2_w'], dec2['fc2_w']),
        jnp.concatenate([dec1['fc2_b'], dec2['fc2_b']]).reshape(1, -1),
        jnp.concatenate([norm1_g, norm2_g]).reshape(1, -1),
        jnp.concatenate([norm1_b, norm2_b]).reshape(1, -1),
        w12, fc3_b.reshape(1, 1),
    ]

    grid = R // _BLK_R
    in_specs = [pl.BlockSpec((_BLK_S, N, F), lambda i: (i, 0, 0)),
                pl.BlockSpec((_BLK_S, N, F), lambda i: (i, 0, 0))]
    in_specs += [pl.BlockSpec(w.shape, lambda i: (0, 0)) for w in weights]
    out_shape = [jax.ShapeDtypeStruct((R, _P), jnp.float32),
                 jax.ShapeDtypeStruct((B, 1), jnp.float32)]
    out_specs = [pl.BlockSpec((_BLK_R, _P), lambda i: (i, 0)),
                 pl.BlockSpec((_BLK_S, 1), lambda i: (i, 0))]

    g, out = pl.pallas_call(
        _body, grid=(grid,), in_specs=in_specs, out_specs=out_specs,
        out_shape=out_shape)(feature1, feature2, *weights)
    return (out.reshape(B), g.reshape(B, N * _P))


# speedup vs baseline: 6.1436x; 1.0291x over previous
import functools

import jax
import jax.numpy as jnp
from jax.experimental import pallas as pl

_N = 4
_D = 64
_H = 4
_HD = _D // _H
_SCALE = _HD ** -0.5
_P = 2 * _D

_BLK_S = 256
_BLK_R = _BLK_S * _N


def _dot(a, b):
    return jnp.dot(a, b, preferred_element_type=jnp.float32)


def _group_shift(x, delta, rowmod):
    if delta == 0:
        return x
    a = jnp.roll(x, -delta, axis=0)
    b = jnp.roll(x, _N - delta, axis=0)
    return jnp.where(rowmod < (_N - delta), a, b)


def _body(x1_ref, x2_ref, wd_ref, bd_ref, wt_ref, bt_ref,
          wk_ref, wv_ref, wq_ref, bq_ref, pw_ref, pb_ref,
          f1w_ref, f1b_ref, f2w_ref, f2b_ref,
          gf_ref, bf_ref, w12_ref, fc3b_ref,
          g_ref, out_ref):
    rowmod = jax.lax.broadcasted_iota(jnp.int32, (_BLK_R, 1), 0) % _N
    ci = jax.lax.broadcasted_iota(jnp.int32, (_P, _P), 0)
    cj = jax.lax.broadcasted_iota(jnp.int32, (_P, _P), 1)
    gmat = jnp.where(ci // _HD == cj // _HD, _SCALE, 0.0).astype(jnp.float32)
    mmat = jnp.where(ci // _D == cj // _D, 1.0 / _D, 0.0).astype(jnp.float32)

    def ln_norm(y):
        m = _dot(y, mmat)
        s2 = _dot(y * y, mmat)
        return (y - m) * jax.lax.rsqrt(s2 - m * m + 1e-6)

    x1 = x1_ref[...].reshape(_BLK_R, x1_ref.shape[-1])
    x2 = x2_ref[...].reshape(_BLK_R, x2_ref.shape[-1])
    f1 = jax.nn.relu(_dot(x1, wd_ref[...]) + bd_ref[...])
    f2 = jax.nn.relu(_dot(x2, wt_ref[...]) + bt_ref[...])
    fp = jnp.concatenate([f1, f2], axis=1)
    y = jnp.concatenate([f2, f1], axis=1)

    k = _dot(fp, wk_ref[...])
    v = _dot(fp, wv_ref[...])
    kd = [_group_shift(k, d, rowmod) for d in range(_N)]
    vd = [_group_shift(v, d, rowmod) for d in range(_N)]

    wq, bq = wq_ref[...], bq_ref[...]
    pw, pb = pw_ref[...], pb_ref[...]
    f1w, f1b = f1w_ref[...], f1b_ref[...]
    f2w, f2b = f2w_ref[...], f2b_ref[...]
    for _ in range(4):
        xn = ln_norm(y)
        q = _dot(xn, wq) + bq
        s = [_dot(q * kd[d], gmat) for d in range(_N)]
        m = functools.reduce(jnp.maximum, s)
        e = [jnp.exp(si - m) for si in s]
        inv = 1.0 / (e[0] + e[1] + e[2] + e[3])
        att = (e[0] * vd[0] + e[1] * vd[1] + e[2] * vd[2] + e[3] * vd[3]) * inv
        ca = _dot(att, pw) + pb
        u = _dot(xn, f1w) + f1b
        h = 0.5 * u * (1.0 + jax.lax.erf(u * 0.7071067811865476))
        y = y + ca + (_dot(h, f2w) + f2b)

    o = ln_norm(y) * gf_ref[...] + bf_ref[...]

    o3 = o.reshape(_BLK_S, _N, _P)
    ot = [o3[:, t, :] for t in range(_N)]
    g_ref[...] = jnp.concatenate(ot, axis=1)
    w12 = w12_ref[...]
    zs = sum(_dot(ot[t], w12[:, t:t + 1]) for t in range(_N))
    out_ref[...] = jax.nn.sigmoid(zs + fc3b_ref[0, 0])


def _blockdiag(a, b):
    ra, ca = a.shape
    rb, cb = b.shape
    return jnp.concatenate([
        jnp.concatenate([a, jnp.zeros((ra, cb), a.dtype)], axis=1),
        jnp.concatenate([jnp.zeros((rb, ca), b.dtype), b], axis=1)], axis=0)


def kernel(feature1, feature2, fc_d_w, fc_d_b, fc_t_w, fc_t_b, dec1, dec2,
           norm1_g, norm1_b, norm2_g, norm2_b, fc3_w, fc3_b):
    B, N, F = feature1.shape
    R = B * N

    def slices(p):
        qkv = p['qkv_w']
        return qkv[:, :_D], qkv[:, _D:2 * _D], qkv[:, 2 * _D:]

    wq1, wk1, wv1 = slices(dec1)
    wq2, wk2, wv2 = slices(dec2)
    wq1f = dec1['n1_g'][:, None] * wq1
    wq2f = dec2['n1_g'][:, None] * wq2
    bq = jnp.concatenate([dec1['n1_b'] @ wq1, dec2['n1_b'] @ wq2])
    f1w1 = dec1['n2_g'][:, None] * dec1['fc1_w']
    f1w2 = dec2['n2_g'][:, None] * dec2['fc1_w']
    f1b = jnp.concatenate([dec1['n2_b'] @ dec1['fc1_w'] + dec1['fc1_b'],
                           dec2['n2_b'] @ dec2['fc1_w'] + dec2['fc1_b']])

    fc3r = fc3_w.reshape(N, 2, _D)
    w12 = jnp.concatenate([fc3r[:, 0, :].T, fc3r[:, 1, :].T], axis=0)

    weights = [
        fc_d_w, fc_d_b.reshape(1, -1), fc_t_w, fc_t_b.reshape(1, -1),
        _blockdiag(wk1, wk2), _blockdiag(wv1, wv2),
        _blockdiag(wq1f, wq2f), bq.reshape(1, -1),
        _blockdiag(dec1['proj_w'], dec2['proj_w']),
        jnp.concatenate([dec1['proj_b'], dec2['proj_b']]).reshape(1, -1),
        _blockdiag(f1w1, f1w2), f1b.reshape(1, -1),
        _blockdiag(dec1['fc2_w'], dec2['fc2_w']),
        jnp.concatenate([dec1['fc2_b'], dec2['fc2_b']]).reshape(1, -1),
        jnp.concatenate([norm1_g, norm2_g]).reshape(1, -1),
        jnp.concatenate([norm1_b, norm2_b]).reshape(1, -1),
        w12, fc3_b.reshape(1, 1),
    ]

    grid = R // _BLK_R
    in_specs = [pl.BlockSpec((_BLK_S, N, F), lambda i: (i, 0, 0)),
                pl.BlockSpec((_BLK_S, N, F), lambda i: (i, 0, 0))]
    in_specs += [pl.BlockSpec(w.shape, lambda i: (0, 0)) for w in weights]
    out_shape = [jax.ShapeDtypeStruct((B, N * _P), jnp.float32),
                 jax.ShapeDtypeStruct((B, 1), jnp.float32)]
    out_specs = [pl.BlockSpec((_BLK_S, N * _P), lambda i: (i, 0)),
                 pl.BlockSpec((_BLK_S, 1), lambda i: (i, 0))]

    g, out = pl.pallas_call(
        _body, grid=(grid,), in_specs=in_specs, out_specs=out_specs,
        out_shape=out_shape)(feature1, feature2, *weights)
    return (out.reshape(B), g)


# speedup vs baseline: 8.1887x; 1.3329x over previous
import functools

import jax
import jax.numpy as jnp
from jax.experimental import pallas as pl

_N = 4
_D = 64
_H = 4
_HD = _D // _H
_SCALE = _HD ** -0.5
_P = 2 * _D

_SG = 4
_LR = _SG * _N


def _dot(a, b):
    return jnp.dot(a, b, preferred_element_type=jnp.float32)


def _body(x1_ref, x2_ref, wd_ref, bd_ref, wt_ref, bt_ref,
          wk_ref, wv_ref, wq_ref, bq_ref, pw_ref, pb_ref,
          f1w_ref, f1b_ref, f2w_ref, f2b_ref,
          gf_ref, bf_ref, w12_ref, fc3b_ref,
          g_ref, out_ref):
    ci = jax.lax.broadcasted_iota(jnp.int32, (_P, _P), 0)
    cj = jax.lax.broadcasted_iota(jnp.int32, (_P, _P), 1)
    gmat = jnp.where(ci // _HD == cj // _HD, _SCALE, 0.0).astype(jnp.float32)
    mmat = jnp.where(ci // _D == cj // _D, 1.0 / _D, 0.0).astype(jnp.float32)

    def ln_norm(y):
        m = _dot(mmat, y)
        s2 = _dot(mmat, y * y)
        return (y - m) * jax.lax.rsqrt(s2 - m * m + 1e-6)

    xt1 = jnp.swapaxes(x1_ref[...], 0, 1)
    xt2 = jnp.swapaxes(x2_ref[...], 0, 1)
    wd, bd = wd_ref[...], bd_ref[...]
    wt, bt = wt_ref[...], bt_ref[...]
    f1 = [jax.nn.relu(_dot(wd, xt1[r]) + bd) for r in range(_LR)]
    f2 = [jax.nn.relu(_dot(wt, xt2[r]) + bt) for r in range(_LR)]
    fp = [jnp.concatenate([f1[r], f2[r]], axis=0) for r in range(_LR)]
    y = [jnp.concatenate([f2[r], f1[r]], axis=0) for r in range(_LR)]

    wk, wv = wk_ref[...], wv_ref[...]
    k = [_dot(wk, fp[r]) for r in range(_LR)]
    v = [_dot(wv, fp[r]) for r in range(_LR)]
    kd = [[k[(r // _N) * _N + (r + d) % _N] for r in range(_LR)]
          for d in range(_N)]
    vd = [[v[(r // _N) * _N + (r + d) % _N] for r in range(_LR)]
          for d in range(_N)]

    wq, bq = wq_ref[...], bq_ref[...]
    pw, pb = pw_ref[...], pb_ref[...]
    f1w, f1b = f1w_ref[...], f1b_ref[...]
    f2w, f2b = f2w_ref[...], f2b_ref[...]
    for _ in range(4):
        yn = [ln_norm(y[r]) for r in range(_LR)]
        q = [_dot(wq, yn[r]) + bq for r in range(_LR)]
        ynew = []
        for r in range(_LR):
            s = [_dot(gmat, q[r] * kd[d][r]) for d in range(_N)]
            m = functools.reduce(jnp.maximum, s)
            e = [jnp.exp(si - m) for si in s]
            inv = 1.0 / (e[0] + e[1] + e[2] + e[3])
            att = (e[0] * vd[0][r] + e[1] * vd[1][r]
                   + e[2] * vd[2][r] + e[3] * vd[3][r]) * inv
            ca = _dot(pw, att) + pb
            u = _dot(f1w, yn[r]) + f1b
            h = 0.5 * u * (1.0 + jax.lax.erf(u * 0.7071067811865476))
            ynew.append(y[r] + ca + (_dot(f2w, h) + f2b))
        y = ynew

    gf, bf = gf_ref[...], bf_ref[...]
    o = [ln_norm(y[r]) * gf + bf for r in range(_LR)]

    ot = [jnp.swapaxes(o[r], 0, 1) for r in range(_LR)]
    g_ref[...] = jnp.concatenate(
        [jnp.concatenate([ot[g * _N + t] for t in range(_N)], axis=1)
         for g in range(_SG)], axis=0)

    w12, b3 = w12_ref[...], fc3b_ref[0, 0]
    z = [_dot(w12[r % _N:r % _N + 1, :], o[r]) for r in range(_LR)]
    out_ref[...] = jnp.concatenate(
        [jax.nn.sigmoid(sum(z[g * _N + t] for t in range(_N)) + b3)
         for g in range(_SG)], axis=0)[None]


def _blockdiag(a, b):
    ra, ca = a.shape
    rb, cb = b.shape
    return jnp.concatenate([
        jnp.concatenate([a, jnp.zeros((ra, cb), a.dtype)], axis=1),
        jnp.concatenate([jnp.zeros((rb, ca), b.dtype), b], axis=1)], axis=0)


def kernel(feature1, feature2, fc_d_w, fc_d_b, fc_t_w, fc_t_b, dec1, dec2,
           norm1_g, norm1_b, norm2_g, norm2_b, fc3_w, fc3_b):
    B, N, F = feature1.shape
    SH = B // 128
    xv1 = feature1.reshape(SH, 128, N, F).transpose(3, 0, 2, 1)
    xv1 = xv1.reshape(F, SH * N, 128)
    xv2 = feature2.reshape(SH, 128, N, F).transpose(3, 0, 2, 1)
    xv2 = xv2.reshape(F, SH * N, 128)

    def slices(p):
        qkv = p['qkv_w']
        return qkv[:, :_D], qkv[:, _D:2 * _D], qkv[:, 2 * _D:]

    wq1, wk1, wv1 = slices(dec1)
    wq2, wk2, wv2 = slices(dec2)
    wq1f = dec1['n1_g'][:, None] * wq1
    wq2f = dec2['n1_g'][:, None] * wq2
    bq = jnp.concatenate([dec1['n1_b'] @ wq1, dec2['n1_b'] @ wq2])
    f1w1 = dec1['n2_g'][:, None] * dec1['fc1_w']
    f1w2 = dec2['n2_g'][:, None] * dec2['fc1_w']
    f1b = jnp.concatenate([dec1['n2_b'] @ dec1['fc1_w'] + dec1['fc1_b'],
                           dec2['n2_b'] @ dec2['fc1_w'] + dec2['fc1_b']])

    fc3r = fc3_w.reshape(N, 2, _D)
    w12t = jnp.concatenate([fc3r[:, 0, :], fc3r[:, 1, :]], axis=1)

    weights = [
        fc_d_w.T, fc_d_b.reshape(-1, 1), fc_t_w.T, fc_t_b.reshape(-1, 1),
        _blockdiag(wk1, wk2).T, _blockdiag(wv1, wv2).T,
        _blockdiag(wq1f, wq2f).T, bq.reshape(-1, 1),
        _blockdiag(dec1['proj_w'], dec2['proj_w']).T,
        jnp.concatenate([dec1['proj_b'], dec2['proj_b']]).reshape(-1, 1),
        _blockdiag(f1w1, f1w2).T, f1b.reshape(-1, 1),
        _blockdiag(dec1['fc2_w'], dec2['fc2_w']).T,
        jnp.concatenate([dec1['fc2_b'], dec2['fc2_b']]).reshape(-1, 1),
        jnp.concatenate([norm1_g, norm2_g]).reshape(-1, 1),
        jnp.concatenate([norm1_b, norm2_b]).reshape(-1, 1),
        w12t, fc3_b.reshape(1, 1),
    ]

    grid = SH // _SG
    in_specs = [pl.BlockSpec((F, _LR, 128), lambda i: (0, i, 0)),
                pl.BlockSpec((F, _LR, 128), lambda i: (0, i, 0))]
    in_specs += [pl.BlockSpec(w.shape, lambda i: (0, 0)) for w in weights]
    out_shape = [jax.ShapeDtypeStruct((B, N * _P), jnp.float32),
                 jax.ShapeDtypeStruct((grid, _SG, 128), jnp.float32)]
    out_specs = [pl.BlockSpec((_SG * 128, N * _P), lambda i: (i, 0)),
                 pl.BlockSpec((1, _SG, 128), lambda i: (i, 0, 0))]

    g, out = pl.pallas_call(
        _body, grid=(grid,), in_specs=in_specs, out_specs=out_specs,
        out_shape=out_shape)(xv1, xv2, *weights)
    return (out.reshape(B), g)
